# Initial kernel scaffold; baseline (speedup 1.0000x reference)
import jax
import jax.numpy as jnp
from jax.experimental import pallas as pl


def kernel(uv_coords, tex0, tex1, tex2):
    raise NotImplementedError("write your pallas kernel here")


# speedup vs baseline: 1.0052x; 1.0052x over previous
import functools

import jax
import jax.numpy as jnp
from jax import lax
from jax.experimental import pallas as pl
from jax.experimental.pallas import tpu as pltpu
from jax.experimental.pallas import tpu_sc as plsc

N = 1048576
LANES = 16
B = 512
NG = B // LANES
IDX_CHUNK = 128
NSEG = 2 * B // IDX_CHUNK

_DEGS = ((0, 3, 2048), (1, 9, 1024), (2, 15, 512))

_COLMAP = []
for _c in range(3):
    _COLMAP.append((0, _c, _c * 9 + 0))
    for _k in range(3):
        _COLMAP.append((1, 3 * _c + _k, _c * 9 + 1 + _k))
    for _k in range(5):
        _COLMAP.append((2, 5 * _c + _k, _c * 9 + 4 + _k))


def kernel(uv_coords, tex0, tex1, tex2):
    tabs = [tex0.reshape(-1, 16), tex1.reshape(-1, 16), tex2.reshape(-1, 16)]
    maxrow = [t.shape[0] - 1 for t in tabs]
    uv_flat = uv_coords.reshape(-1)

    info = plsc.get_sparse_core_info()
    nc, ns = info.num_cores, info.num_subcores
    nw = nc * ns
    pts_per_tile = N // nw
    n_chunks = pts_per_tile // B

    @functools.partial(
        pl.kernel,
        out_type=jax.ShapeDtypeStruct((N * 27,), jnp.float32),
        mesh=plsc.VectorSubcoreMesh(core_axis_name="c", subcore_axis_name="s"),
        compiler_params=pltpu.CompilerParams(
            needs_layout_passes=False, use_tc_tiling_on_sc=False),
        scratch_types=[
            pltpu.VMEM((2 * B,), jnp.float32),
            pltpu.VMEM((2 * B,), jnp.int32),
            pltpu.VMEM((2 * B,), jnp.int32),
            pltpu.VMEM((2 * B,), jnp.int32),
            pltpu.VMEM((B,), jnp.int32),
            pltpu.VMEM((B,), jnp.int32),
            pltpu.VMEM((B,), jnp.int32),
            pltpu.VMEM((2 * B, 16), jnp.float32),
            pltpu.VMEM((2 * B, 16), jnp.float32),
            pltpu.VMEM((2 * B, 16), jnp.float32),
            pltpu.VMEM((B * 27,), jnp.float32),
            pltpu.SemaphoreType.DMA,
        ],
    )
    def sc_kernel(uv_hbm, t0_hbm, t1_hbm, t2_hbm, out_hbm,
                  uv_v, i0_v, i1_v, i2_v, bo0_v, bo1_v, bo2_v,
                  g0_v, g1_v, g2_v, out_v, sem):
        wid = lax.axis_index("s") * nc + lax.axis_index("c")
        iota = lax.iota(jnp.int32, LANES)
        idx_refs = (i0_v, i1_v, i2_v)
        bo_refs = (bo0_v, bo1_v, bo2_v)
        g_refs = (g0_v, g1_v, g2_v)

        def chunk_body(ci, _):
            base = wid * pts_per_tile + ci * B
            pltpu.sync_copy(uv_hbm.at[pl.ds(2 * base, 2 * B)], uv_v)

            def gen_body(g, _):
                p = iota + g * LANES
                p2 = p << 1
                u = plsc.load_gather(uv_v, [p2])
                v = plsc.load_gather(uv_v, [p2 + 1])
                ix = jnp.clip((u * 2048.0).astype(jnp.int32), 0, 2047)
                iy = jnp.clip((v * 2048.0).astype(jnp.int32), 0, 2047)
                for s, d, w in _DEGS:
                    sh = s
                    texel = ((iy >> sh) << (11 - sh)) + (ix >> sh)
                    o = texel * d
                    r = o >> 4
                    r2 = jnp.minimum(r + 1, maxrow[s])
                    plsc.store_scatter(idx_refs[s], [p2], r)
                    plsc.store_scatter(idx_refs[s], [p2 + 1], r2)
                    plsc.store_scatter(bo_refs[s], [p], (p << 5) + (o & 15))
                return _

            lax.fori_loop(0, NG, gen_body, None)

            copies = []
            for s in range(3):
                tab = (t0_hbm, t1_hbm, t2_hbm)[s]
                for i in range(NSEG):
                    sl = pl.ds(i * IDX_CHUNK, IDX_CHUNK)
                    copies.append(pltpu.async_copy(
                        tab.at[idx_refs[s].at[sl]],
                        g_refs[s].at[sl], sem))
            for c in copies:
                c.wait()

            def shuf_body(g, _):
                p = iota + g * LANES
                p27 = p * 27
                bo = [plsc.load_gather(bo_refs[s], [p]) for s in range(3)]
                for s, col, j in _COLMAP:
                    w = bo[s] + col
                    val = plsc.load_gather(g_refs[s], [w >> 4, w & 15])
                    plsc.store_scatter(out_v, [p27 + j], val)
                return _

            lax.fori_loop(0, NG, shuf_body, None)
            pltpu.sync_copy(out_v, out_hbm.at[pl.ds(base * 27, B * 27)])
            return _

        lax.fori_loop(0, n_chunks, chunk_body, None)

    out = sc_kernel(uv_flat, *tabs)
    return out.reshape(N, 3, 9)
